# Initial kernel scaffold; baseline (speedup 1.0000x reference)
import jax
import jax.numpy as jnp
from jax.experimental import pallas as pl


def kernel(token_ids, weight):
    raise NotImplementedError("write your pallas kernel here")


# speedup vs baseline: 1.6823x; 1.6823x over previous
import functools

import jax
import jax.numpy as jnp
from jax import lax
from jax.experimental import pallas as pl
from jax.experimental.pallas import tpu as pltpu
from jax.experimental.pallas import tpu_sc as plsc

NUM_EMBEDDING = 1000000
EMBEDDING_DIM = 64

_INFO = plsc.get_sparse_core_info()
_NC = _INFO.num_cores
_NS = _INFO.num_subcores
_NW = _NC * _NS
_CHUNK = 128


def _make_gather(total, chunks_per_w):
    b_per_w = chunks_per_w * _CHUNK
    mesh = plsc.VectorSubcoreMesh(core_axis_name="c", subcore_axis_name="s")

    @functools.partial(
        pl.kernel,
        mesh=mesh,
        out_type=jax.ShapeDtypeStruct((total, EMBEDDING_DIM), jnp.float32),
        scratch_types=[
            pltpu.VMEM((chunks_per_w, _CHUNK), jnp.int32),
            pltpu.VMEM((_CHUNK, EMBEDDING_DIM), jnp.float32),
            pltpu.SemaphoreType.DMA,
        ],
        compiler_params=pltpu.CompilerParams(use_tc_tiling_on_sc=False),
    )
    def gather_kernel(idx_hbm, table_hbm, out_hbm, idx_v, rows_v, sem):
        wid = lax.axis_index("s") * _NC + lax.axis_index("c")
        pltpu.sync_copy(idx_hbm.at[wid], idx_v)
        base = wid * b_per_w

        def chunk_body(j, carry):
            pltpu.async_copy(table_hbm.at[idx_v.at[j]], rows_v, sem).wait()
            pltpu.sync_copy(rows_v, out_hbm.at[pl.ds(base + j * _CHUNK, _CHUNK)])
            return carry

        lax.fori_loop(0, chunks_per_w, chunk_body, 0)

    return gather_kernel


def kernel(token_ids, weight):
    n_tokens, n_per = token_ids.shape
    total = n_tokens * n_per
    chunks_per_w = total // (_NW * _CHUNK)
    idx = token_ids.reshape(_NW, chunks_per_w, _CHUNK).astype(jnp.int32)
    out = _make_gather(total, chunks_per_w)(idx, weight)
    return out.reshape(n_tokens, n_per, EMBEDDING_DIM)


# speedup vs baseline: 1.8674x; 1.1100x over previous
import functools

import jax
import jax.numpy as jnp
from jax import lax
from jax.experimental import pallas as pl
from jax.experimental.pallas import tpu as pltpu
from jax.experimental.pallas import tpu_sc as plsc

NUM_EMBEDDING = 1000000
EMBEDDING_DIM = 64

_INFO = plsc.get_sparse_core_info()
_NC = _INFO.num_cores
_NS = _INFO.num_subcores
_NW = _NC * _NS
_CHUNK = 128


_K = 4
_SUPER = _K * _CHUNK


def _make_gather(total, chunks_per_w):
    b_per_w = chunks_per_w * _CHUNK
    n_super = chunks_per_w // _K
    mesh = plsc.VectorSubcoreMesh(core_axis_name="c", subcore_axis_name="s")

    @functools.partial(
        pl.kernel,
        mesh=mesh,
        out_type=jax.ShapeDtypeStruct((total, EMBEDDING_DIM), jnp.float32),
        scratch_types=[
            pltpu.VMEM((chunks_per_w, _CHUNK), jnp.int32),
            pltpu.VMEM((2, _SUPER, EMBEDDING_DIM), jnp.float32),
            pltpu.SemaphoreType.DMA,
            pltpu.SemaphoreType.DMA,
        ],
        compiler_params=pltpu.CompilerParams(use_tc_tiling_on_sc=False),
    )
    def gather_kernel(idx_hbm, table_hbm, out_hbm, idx_v, rows_v, gsem, wsem):
        wid = lax.axis_index("s") * _NC + lax.axis_index("c")
        pltpu.sync_copy(idx_hbm.at[wid], idx_v)
        base = wid * b_per_w

        def fire_gathers(super_i, buf):
            for k in range(_K):
                pltpu.async_copy(
                    table_hbm.at[idx_v.at[super_i * _K + k]],
                    rows_v.at[buf, pl.ds(k * _CHUNK, _CHUNK)],
                    gsem,
                )

        def drain_gathers(buf):
            pltpu.make_async_copy(
                out_hbm.at[pl.ds(base, _SUPER)], rows_v.at[buf], gsem
            ).wait()

        def drain_one_writeback():
            pltpu.make_async_copy(
                rows_v.at[0], out_hbm.at[pl.ds(base, _SUPER)], wsem
            ).wait()

        fire_gathers(0, 0)
        pltpu.async_copy(
            rows_v.at[1], out_hbm.at[pl.ds(base, _SUPER)], wsem
        )

        def super_body(i, carry):
            cur = lax.rem(i, 2)
            nxt = 1 - cur
            drain_gathers(cur)
            drain_one_writeback()
            nxt_i = lax.min(i + 1, n_super - 1)
            fire_gathers(nxt_i, nxt)
            pltpu.async_copy(
                rows_v.at[cur],
                out_hbm.at[pl.ds(base + i * _SUPER, _SUPER)],
                wsem,
            )
            return carry

        lax.fori_loop(0, n_super, super_body, 0)
        drain_gathers(lax.rem(n_super, 2))
        drain_one_writeback()

    return gather_kernel


def kernel(token_ids, weight):
    n_tokens, n_per = token_ids.shape
    total = n_tokens * n_per
    chunks_per_w = total // (_NW * _CHUNK)
    idx = token_ids.reshape(_NW, chunks_per_w, _CHUNK).astype(jnp.int32)
    out = _make_gather(total, chunks_per_w)(idx, weight)
    return out.reshape(n_tokens, n_per, EMBEDDING_DIM)
